# baseline (speedup 1.0000x reference)
import functools

import jax
import jax.numpy as jnp
from jax import lax
from jax.experimental import pallas as pl
from jax.experimental.pallas import tpu as pltpu
from jax.experimental.pallas import tpu_sc as plsc

_NC = 2
_NS = 16
_NW = _NC * _NS
_CHUNK = 128

_F32 = jnp.float32

_CORE0_SHARE = 0.25


def _split(total):
    n0 = int(round(total * _CORE0_SHARE / 2)) * 2
    n0 = min(max(n0, 2), total - 2)
    return n0, total - n0


def _pad_idx_split(idx, pad_value, n0, n1):
    idx = idx.astype(jnp.int32)
    total = _NS * (n0 + n1) * _CHUNK
    idx = jnp.concatenate(
        [idx, jnp.full((total - idx.shape[0],), pad_value, jnp.int32)])
    chunks = idx.reshape(_NS * (n0 + n1), _CHUNK)
    g0 = _NS * n0
    c0 = chunks[:g0].reshape(_NS, n0, _CHUNK)
    c1 = chunks[g0:].reshape(_NS, n1, _CHUNK)
    maxc = max(n0, n1)
    c0 = jnp.pad(c0, ((0, 0), (0, maxc - n0), (0, 0)))
    c1 = jnp.pad(c1, ((0, 0), (0, maxc - n1), (0, 0)))
    return jnp.stack([c0, c1])


def _sc_degree(dst4, ones_hbm, zeros1, n_acc, n0, n1):
    rows = n_acc // _NS
    maxc = max(n0, n1)
    mesh = plsc.VectorSubcoreMesh(core_axis_name="c", subcore_axis_name="s")

    @functools.partial(
        pl.kernel,
        out_type=jax.ShapeDtypeStruct((_NC, n_acc), _F32),
        mesh=mesh,
        scratch_types=[
            pltpu.VMEM((maxc, _CHUNK), jnp.int32),
            pltpu.VMEM((_CHUNK,), _F32),
            pltpu.VMEM_SHARED((n_acc,), _F32),
        ],
        compiler_params=pltpu.CompilerParams(use_tc_tiling_on_sc=False),
    )
    def k(dst_hbm, ones_h, zeros_h, out_hbm, idx_v, ones_v, acc):
        cid = lax.axis_index("c")
        sid = lax.axis_index("s")
        n_c = jnp.where(cid == 0, n0, n1)
        pltpu.sync_copy(zeros_h.at[pl.ds(sid * rows, rows)],
                        acc.at[pl.ds(sid * rows, rows)])
        pltpu.sync_copy(ones_h, ones_v)
        pltpu.sync_copy(dst_hbm.at[cid, sid], idx_v)
        plsc.subcore_barrier()

        def body(j, carry):
            pltpu.sync_copy(ones_v, acc.at[idx_v.at[j]], add=True)
            return carry

        lax.fori_loop(0, n_c, body, 0)
        plsc.subcore_barrier()
        pltpu.sync_copy(acc.at[pl.ds(sid * rows, rows)],
                        out_hbm.at[cid, pl.ds(sid * rows, rows)])

    return k(dst4, ones_hbm, zeros1)


def _sc_aggregate(hs, src4, dst4, zeros2, n_acc, n0, n1, h_dim):
    rows = n_acc // _NS
    maxc = max(n0, n1)
    mesh = plsc.VectorSubcoreMesh(core_axis_name="c", subcore_axis_name="s")

    @functools.partial(
        pl.kernel,
        out_type=jax.ShapeDtypeStruct((_NC, n_acc, h_dim), _F32),
        mesh=mesh,
        scratch_types=[
            pltpu.VMEM((maxc, _CHUNK), jnp.int32),
            pltpu.VMEM((maxc, _CHUNK), jnp.int32),
            pltpu.VMEM((_CHUNK, h_dim), _F32),
            pltpu.VMEM((_CHUNK, h_dim), _F32),
            pltpu.SemaphoreType.DMA,
            pltpu.SemaphoreType.DMA,
            pltpu.VMEM_SHARED((n_acc, h_dim), _F32),
        ],
        compiler_params=pltpu.CompilerParams(use_tc_tiling_on_sc=False),
    )
    def k(hs_hbm, src_hbm, dst_hbm, zeros_h, out_hbm,
          idxs_v, idxd_v, rows_a, rows_b, sem_a, sem_b, acc):
        cid = lax.axis_index("c")
        sid = lax.axis_index("s")
        n_c = jnp.where(cid == 0, n0, n1)
        pltpu.sync_copy(zeros_h.at[pl.ds(sid * rows, rows)],
                        acc.at[pl.ds(sid * rows, rows)])
        pltpu.sync_copy(src_hbm.at[cid, sid], idxs_v)
        pltpu.sync_copy(dst_hbm.at[cid, sid], idxd_v)
        plsc.subcore_barrier()

        pltpu.async_copy(hs_hbm.at[idxs_v.at[0]], rows_a, sem_a)

        def body(i, carry):
            j0 = 2 * i
            j1 = j0 + 1
            pltpu.async_copy(hs_hbm.at[idxs_v.at[j1]], rows_b, sem_b)
            pltpu.make_async_copy(hs_hbm.at[idxs_v.at[j0]],
                                  rows_a, sem_a).wait()
            pltpu.sync_copy(rows_a, acc.at[idxd_v.at[j0]], add=True)

            @pl.when(j0 + 2 < n_c)
            def _():
                pltpu.async_copy(hs_hbm.at[idxs_v.at[j0 + 2]], rows_a, sem_a)

            pltpu.make_async_copy(hs_hbm.at[idxs_v.at[j1]],
                                  rows_b, sem_b).wait()
            pltpu.sync_copy(rows_b, acc.at[idxd_v.at[j1]], add=True)
            return carry

        lax.fori_loop(0, n_c // 2, body, 0)
        plsc.subcore_barrier()
        pltpu.sync_copy(acc.at[pl.ds(sid * rows, rows)],
                        out_hbm.at[cid, pl.ds(sid * rows, rows)])

    return k(hs, src4, dst4, zeros2)


def _sc_edge_gather(table, asrc4, atgt4, a_pad, n0, n1, width):
    maxc = max(n0, n1)
    mesh = plsc.VectorSubcoreMesh(core_axis_name="c", subcore_axis_name="s")

    @functools.partial(
        pl.kernel,
        out_type=(jax.ShapeDtypeStruct((a_pad, width), _F32),
                  jax.ShapeDtypeStruct((a_pad, width), _F32)),
        mesh=mesh,
        scratch_types=[
            pltpu.VMEM((maxc, _CHUNK), jnp.int32),
            pltpu.VMEM((maxc, _CHUNK), jnp.int32),
            pltpu.VMEM((_CHUNK, width), _F32),
            pltpu.VMEM((_CHUNK, width), _F32),
            pltpu.VMEM((_CHUNK, width), _F32),
            pltpu.VMEM((_CHUNK, width), _F32),
            pltpu.SemaphoreType.DMA,
            pltpu.SemaphoreType.DMA,
            pltpu.SemaphoreType.DMA,
            pltpu.SemaphoreType.DMA,
        ],
        compiler_params=pltpu.CompilerParams(use_tc_tiling_on_sc=False),
    )
    def k(table_hbm, asrc_hbm, atgt_hbm, outs_hbm, outt_hbm,
          idxs_v, idxt_v, rs_a, rt_a, rs_b, rt_b,
          sem_sa, sem_ta, sem_sb, sem_tb):
        cid = lax.axis_index("c")
        sid = lax.axis_index("s")
        n_c = jnp.where(cid == 0, n0, n1)
        base = jnp.where(cid == 0, sid * n0, _NS * n0 + sid * n1) * _CHUNK
        pltpu.sync_copy(asrc_hbm.at[cid, sid], idxs_v)
        pltpu.sync_copy(atgt_hbm.at[cid, sid], idxt_v)

        pltpu.async_copy(table_hbm.at[idxs_v.at[0]], rs_a, sem_sa)
        pltpu.async_copy(table_hbm.at[idxt_v.at[0]], rt_a, sem_ta)

        def body(i, carry):
            j0 = 2 * i
            j1 = j0 + 1
            row0 = base + j0 * _CHUNK
            row1 = base + j1 * _CHUNK
            pltpu.async_copy(table_hbm.at[idxs_v.at[j1]], rs_b, sem_sb)
            pltpu.async_copy(table_hbm.at[idxt_v.at[j1]], rt_b, sem_tb)
            pltpu.make_async_copy(table_hbm.at[idxs_v.at[j0]],
                                  rs_a, sem_sa).wait()
            pltpu.sync_copy(rs_a, outs_hbm.at[pl.ds(row0, _CHUNK)])
            pltpu.make_async_copy(table_hbm.at[idxt_v.at[j0]],
                                  rt_a, sem_ta).wait()
            pltpu.sync_copy(rt_a, outt_hbm.at[pl.ds(row0, _CHUNK)])

            @pl.when(j0 + 2 < n_c)
            def _():
                pltpu.async_copy(table_hbm.at[idxs_v.at[j0 + 2]], rs_a, sem_sa)
                pltpu.async_copy(table_hbm.at[idxt_v.at[j0 + 2]], rt_a, sem_ta)

            pltpu.make_async_copy(table_hbm.at[idxs_v.at[j1]],
                                  rs_b, sem_sb).wait()
            pltpu.sync_copy(rs_b, outs_hbm.at[pl.ds(row1, _CHUNK)])
            pltpu.make_async_copy(table_hbm.at[idxt_v.at[j1]],
                                  rt_b, sem_tb).wait()
            pltpu.sync_copy(rt_b, outt_hbm.at[pl.ds(row1, _CHUNK)])
            return carry

        lax.fori_loop(0, n_c // 2, body, 0)

    return k(table, asrc4, atgt4)


def _tc_proj1(x, w1, dp0, dp1, n):
    def body(x_ref, w_ref, d0_ref, d1_ref, out_ref):
        deg = d0_ref[0:n, :] + d1_ref[0:n, :] + 1.0
        dinv = lax.rsqrt(deg)
        p = jnp.dot(x_ref[...], w_ref[...], preferred_element_type=_F32)
        out_ref[...] = p * dinv

    return pl.pallas_call(
        body,
        out_shape=jax.ShapeDtypeStruct((n, w1.shape[1]), _F32),
    )(x, w1, dp0, dp1)


def _tc_conv1_epilogue(a0, a1, hs1, dp0, dp1, w2, b1, n):
    def body(a0_ref, a1_ref, hs1_ref, d0_ref, d1_ref, w_ref, b_ref, out_ref):
        deg = d0_ref[0:n, :] + d1_ref[0:n, :] + 1.0
        dinv = lax.rsqrt(deg)
        agg = a0_ref[0:n, :] + a1_ref[0:n, :] + hs1_ref[...]
        h = jnp.maximum(agg * dinv + b_ref[...], 0.0)
        out_ref[...] = jnp.dot(h, w_ref[...], preferred_element_type=_F32) * dinv

    return pl.pallas_call(
        body,
        out_shape=jax.ShapeDtypeStruct((n, w2.shape[1]), _F32),
    )(a0, a1, hs1, dp0, dp1, w2, b1)


def _tc_heads_node(a0, a1, hs2, dp0, dp1, b2, p1, pb1, p2, pb2, armyf, n,
                   width):
    h = hs2.shape[1]
    pad = width - h - 1

    def body(a0_ref, a1_ref, hs2_ref, d0_ref, d1_ref, b2_ref,
             p1_ref, pb1_ref, p2_ref, pb2_ref, army_ref,
             plc_ref, table_ref):
        deg = d0_ref[0:n, :] + d1_ref[0:n, :] + 1.0
        dinv = lax.rsqrt(deg)
        agg = a0_ref[0:n, :] + a1_ref[0:n, :] + hs2_ref[...]
        emb = agg * dinv + b2_ref[...]
        hp = jnp.maximum(
            jnp.dot(emb, p1_ref[...], preferred_element_type=_F32)
            + pb1_ref[...], 0.0)
        plc_ref[...] = (jnp.dot(hp, p2_ref[...], preferred_element_type=_F32)
                        + pb2_ref[...])
        table_ref[...] = jnp.concatenate(
            [emb, army_ref[...], jnp.zeros((n, pad), _F32)], axis=1)

    return pl.pallas_call(
        body,
        out_shape=(jax.ShapeDtypeStruct((n, 1), _F32),
                   jax.ShapeDtypeStruct((n, width), _F32)),
    )(a0, a1, hs2, dp0, dp1, b2, p1, pb1, p2, pb2, armyf)


def _tc_edge_heads(s_tab, t_tab, asrc, atgt, ws, wt, bcat, es2, esb2,
                   as2, asb2, a_total, blk, h, max_army):
    grid = a_total // blk
    width = s_tab.shape[1]
    es_h = es2.shape[0]

    def body(s_ref, t_ref, asrc_ref, atgt_ref, ws_ref, wt_ref, bc_ref,
             es2_ref, esb2_ref, as2_ref, asb2_ref, el_ref, al_ref):
        s_emb = s_ref[:, 0:h]
        t_emb = t_ref[:, 0:h]
        sa = s_ref[:, h:h + 1]
        ta = t_ref[:, h:h + 1]
        hid = jnp.maximum(
            jnp.dot(s_emb, ws_ref[...], preferred_element_type=_F32)
            + jnp.dot(t_emb, wt_ref[...], preferred_element_type=_F32)
            + bc_ref[...], 0.0)
        el = (jnp.dot(hid[:, 0:es_h], es2_ref[...],
                      preferred_element_type=_F32) + esb2_ref[...])
        bad = jnp.logical_or(sa <= 2.0, ta >= 3.0 * sa).astype(_F32)
        selfe = (asrc_ref[...] == atgt_ref[...]).astype(_F32)
        el_ref[...] = el - bad - 100.0 * selfe
        al = (jnp.dot(hid[:, es_h:], as2_ref[...],
                      preferred_element_type=_F32) + asb2_ref[...])
        army_ix = lax.broadcasted_iota(jnp.int32, (blk, max_army), 1).astype(_F32)
        valid = army_ix <= sa - 1.0
        al_ref[...] = jnp.where(valid, al, _F32(-1e9))

    return pl.pallas_call(
        body,
        grid=(grid,),
        in_specs=[
            pl.BlockSpec((blk, width), lambda i: (i, 0)),
            pl.BlockSpec((blk, width), lambda i: (i, 0)),
            pl.BlockSpec((blk, 1), lambda i: (i, 0)),
            pl.BlockSpec((blk, 1), lambda i: (i, 0)),
            pl.BlockSpec(ws.shape, lambda i: (0, 0)),
            pl.BlockSpec(wt.shape, lambda i: (0, 0)),
            pl.BlockSpec(bcat.shape, lambda i: (0,)),
            pl.BlockSpec(es2.shape, lambda i: (0, 0)),
            pl.BlockSpec(esb2.shape, lambda i: (0,)),
            pl.BlockSpec(as2.shape, lambda i: (0, 0)),
            pl.BlockSpec(asb2.shape, lambda i: (0,)),
        ],
        out_specs=(
            pl.BlockSpec((blk, 1), lambda i: (i, 0)),
            pl.BlockSpec((blk, max_army), lambda i: (i, 0)),
        ),
        out_shape=(jax.ShapeDtypeStruct((a_total, 1), _F32),
                   jax.ShapeDtypeStruct((a_total, max_army), _F32)),
    )(s_tab, t_tab, asrc, atgt, ws, wt, bcat, es2, esb2, as2, asb2)


def kernel(x, edge_index, action_edges, army_counts, W1, b1, W2, b2,
           P1, pb1, P2, pb2, ES1, esb1, ES2, esb2, AS1, asb1, AS2, asb2):
    n, d = x.shape
    h = W1.shape[1]
    e = edge_index.shape[1]
    a = action_edges.shape[0]
    max_army = AS2.shape[1]
    width = 80

    n_acc = ((n + _NS * 8) + _NS * 8 - 1) // (_NS * 8) * (_NS * 8)
    ce = -(-e // (_NW * _CHUNK))
    ca = -(-a // (_NW * _CHUNK))
    ce0, ce1 = _split(2 * ce)
    ca0, ca1 = _split(2 * ca)
    a_pad = _NS * (ca0 + ca1) * _CHUNK

    src = edge_index[0].astype(jnp.int32)
    dst = edge_index[1].astype(jnp.int32)
    src3 = _pad_idx_split(src, 0, ce0, ce1)
    dst3 = _pad_idx_split(dst, n, ce0, ce1)
    asrc = action_edges[:, 0].astype(jnp.int32)
    atgt = action_edges[:, 1].astype(jnp.int32)
    asrc3 = _pad_idx_split(asrc, 0, ca0, ca1)
    atgt3 = _pad_idx_split(atgt, 0, ca0, ca1)

    ones_h = jnp.ones((_CHUNK,), _F32)
    zeros1 = jnp.zeros((n_acc,), _F32)
    zeros2 = jnp.zeros((n_acc, h), _F32)
    armyf = army_counts.astype(_F32).reshape(n, 1)

    degp = _sc_degree(dst3, ones_h, zeros1, n_acc, ce0, ce1)
    dp0 = degp[0].reshape(n_acc, 1)
    dp1 = degp[1].reshape(n_acc, 1)
    hs1 = _tc_proj1(x, W1, dp0, dp1, n)

    agg1 = _sc_aggregate(hs1, src3, dst3, zeros2, n_acc, ce0, ce1, h)
    hs2 = _tc_conv1_epilogue(agg1[0], agg1[1], hs1, dp0, dp1, W2, b1, n)

    agg2 = _sc_aggregate(hs2, src3, dst3, zeros2, n_acc, ce0, ce1, h)
    plc, table = _tc_heads_node(agg2[0], agg2[1], hs2, dp0, dp1, b2,
                                P1, pb1, P2, pb2, armyf, n, width)

    s_tab, t_tab = _sc_edge_gather(table, asrc3, atgt3, a_pad, ca0, ca1,
                                   width)
    ws = jnp.concatenate([ES1[:h], AS1[:h]], axis=1)
    wt = jnp.concatenate([ES1[h:], AS1[h:]], axis=1)
    bcat = jnp.concatenate([esb1, asb1])
    el, al = _tc_edge_heads(s_tab, t_tab, asrc.reshape(a, 1),
                            atgt.reshape(a, 1), ws, wt, bcat, ES2, esb2,
                            AS2, asb2, a, 2000, h, max_army)

    return plc.reshape(n), el.reshape(a), al


# speedup vs baseline: 1.0373x; 1.0373x over previous
import functools

import jax
import jax.numpy as jnp
from jax import lax
from jax.experimental import pallas as pl
from jax.experimental.pallas import tpu as pltpu
from jax.experimental.pallas import tpu_sc as plsc

_NC = 2
_NS = 16
_NW = _NC * _NS
_CHUNK = 128

_F32 = jnp.float32

_CORE0_SHARE = 0.75


def _split(total):
    n0 = int(round(total * _CORE0_SHARE / 2)) * 2
    n0 = min(max(n0, 2), total - 2)
    return n0, total - n0


def _pad_idx_split(idx, pad_value, n0, n1):
    idx = idx.astype(jnp.int32)
    total = _NS * (n0 + n1) * _CHUNK
    idx = jnp.concatenate(
        [idx, jnp.full((total - idx.shape[0],), pad_value, jnp.int32)])
    chunks = idx.reshape(_NS * (n0 + n1), _CHUNK)
    g0 = _NS * n0
    c0 = chunks[:g0].reshape(_NS, n0, _CHUNK)
    c1 = chunks[g0:].reshape(_NS, n1, _CHUNK)
    maxc = max(n0, n1)
    c0 = jnp.pad(c0, ((0, 0), (0, maxc - n0), (0, 0)))
    c1 = jnp.pad(c1, ((0, 0), (0, maxc - n1), (0, 0)))
    return jnp.stack([c0, c1])


def _sc_degree(dst4, ones_hbm, zeros1, n_acc, n0, n1):
    rows = n_acc // _NS
    maxc = max(n0, n1)
    mesh = plsc.VectorSubcoreMesh(core_axis_name="c", subcore_axis_name="s")

    @functools.partial(
        pl.kernel,
        out_type=jax.ShapeDtypeStruct((_NC, n_acc), _F32),
        mesh=mesh,
        scratch_types=[
            pltpu.VMEM((maxc, _CHUNK), jnp.int32),
            pltpu.VMEM((_CHUNK,), _F32),
            pltpu.VMEM_SHARED((n_acc,), _F32),
        ],
        compiler_params=pltpu.CompilerParams(use_tc_tiling_on_sc=False),
    )
    def k(dst_hbm, ones_h, zeros_h, out_hbm, idx_v, ones_v, acc):
        cid = lax.axis_index("c")
        sid = lax.axis_index("s")
        n_c = jnp.where(cid == 0, n0, n1)
        pltpu.sync_copy(zeros_h.at[pl.ds(sid * rows, rows)],
                        acc.at[pl.ds(sid * rows, rows)])
        pltpu.sync_copy(ones_h, ones_v)
        pltpu.sync_copy(dst_hbm.at[cid, sid], idx_v)
        plsc.subcore_barrier()

        def body(j, carry):
            pltpu.sync_copy(ones_v, acc.at[idx_v.at[j]], add=True)
            return carry

        lax.fori_loop(0, n_c, body, 0)
        plsc.subcore_barrier()
        pltpu.sync_copy(acc.at[pl.ds(sid * rows, rows)],
                        out_hbm.at[cid, pl.ds(sid * rows, rows)])

    return k(dst4, ones_hbm, zeros1)


def _sc_aggregate(hs, src4, dst4, zeros2, n_acc, n0, n1, h_dim):
    rows = n_acc // _NS
    maxc = max(n0, n1)
    mesh = plsc.VectorSubcoreMesh(core_axis_name="c", subcore_axis_name="s")

    @functools.partial(
        pl.kernel,
        out_type=jax.ShapeDtypeStruct((_NC, n_acc, h_dim), _F32),
        mesh=mesh,
        scratch_types=[
            pltpu.VMEM((maxc, _CHUNK), jnp.int32),
            pltpu.VMEM((maxc, _CHUNK), jnp.int32),
            pltpu.VMEM((_CHUNK, h_dim), _F32),
            pltpu.VMEM((_CHUNK, h_dim), _F32),
            pltpu.SemaphoreType.DMA,
            pltpu.SemaphoreType.DMA,
            pltpu.VMEM_SHARED((n_acc, h_dim), _F32),
        ],
        compiler_params=pltpu.CompilerParams(use_tc_tiling_on_sc=False),
    )
    def k(hs_hbm, src_hbm, dst_hbm, zeros_h, out_hbm,
          idxs_v, idxd_v, rows_a, rows_b, sem_a, sem_b, acc):
        cid = lax.axis_index("c")
        sid = lax.axis_index("s")
        n_c = jnp.where(cid == 0, n0, n1)
        pltpu.sync_copy(zeros_h.at[pl.ds(sid * rows, rows)],
                        acc.at[pl.ds(sid * rows, rows)])
        pltpu.sync_copy(src_hbm.at[cid, sid], idxs_v)
        pltpu.sync_copy(dst_hbm.at[cid, sid], idxd_v)
        plsc.subcore_barrier()

        pltpu.async_copy(hs_hbm.at[idxs_v.at[0]], rows_a, sem_a)

        def body(i, carry):
            j0 = 2 * i
            j1 = j0 + 1
            pltpu.async_copy(hs_hbm.at[idxs_v.at[j1]], rows_b, sem_b)
            pltpu.make_async_copy(hs_hbm.at[idxs_v.at[j0]],
                                  rows_a, sem_a).wait()
            pltpu.sync_copy(rows_a, acc.at[idxd_v.at[j0]], add=True)

            @pl.when(j0 + 2 < n_c)
            def _():
                pltpu.async_copy(hs_hbm.at[idxs_v.at[j0 + 2]], rows_a, sem_a)

            pltpu.make_async_copy(hs_hbm.at[idxs_v.at[j1]],
                                  rows_b, sem_b).wait()
            pltpu.sync_copy(rows_b, acc.at[idxd_v.at[j1]], add=True)
            return carry

        lax.fori_loop(0, n_c // 2, body, 0)
        plsc.subcore_barrier()
        pltpu.sync_copy(acc.at[pl.ds(sid * rows, rows)],
                        out_hbm.at[cid, pl.ds(sid * rows, rows)])

    return k(hs, src4, dst4, zeros2)


def _sc_edge_gather(table, asrc4, atgt4, a_pad, n0, n1, width):
    maxc = max(n0, n1)
    mesh = plsc.VectorSubcoreMesh(core_axis_name="c", subcore_axis_name="s")

    @functools.partial(
        pl.kernel,
        out_type=(jax.ShapeDtypeStruct((a_pad, width), _F32),
                  jax.ShapeDtypeStruct((a_pad, width), _F32)),
        mesh=mesh,
        scratch_types=[
            pltpu.VMEM((maxc, _CHUNK), jnp.int32),
            pltpu.VMEM((maxc, _CHUNK), jnp.int32),
            pltpu.VMEM((_CHUNK, width), _F32),
            pltpu.VMEM((_CHUNK, width), _F32),
            pltpu.VMEM((_CHUNK, width), _F32),
            pltpu.VMEM((_CHUNK, width), _F32),
            pltpu.SemaphoreType.DMA,
            pltpu.SemaphoreType.DMA,
            pltpu.SemaphoreType.DMA,
            pltpu.SemaphoreType.DMA,
        ],
        compiler_params=pltpu.CompilerParams(use_tc_tiling_on_sc=False),
    )
    def k(table_hbm, asrc_hbm, atgt_hbm, outs_hbm, outt_hbm,
          idxs_v, idxt_v, rs_a, rt_a, rs_b, rt_b,
          sem_sa, sem_ta, sem_sb, sem_tb):
        cid = lax.axis_index("c")
        sid = lax.axis_index("s")
        n_c = jnp.where(cid == 0, n0, n1)
        base = jnp.where(cid == 0, sid * n0, _NS * n0 + sid * n1) * _CHUNK
        pltpu.sync_copy(asrc_hbm.at[cid, sid], idxs_v)
        pltpu.sync_copy(atgt_hbm.at[cid, sid], idxt_v)

        pltpu.async_copy(table_hbm.at[idxs_v.at[0]], rs_a, sem_sa)
        pltpu.async_copy(table_hbm.at[idxt_v.at[0]], rt_a, sem_ta)

        def body(i, carry):
            j0 = 2 * i
            j1 = j0 + 1
            row0 = base + j0 * _CHUNK
            row1 = base + j1 * _CHUNK
            pltpu.async_copy(table_hbm.at[idxs_v.at[j1]], rs_b, sem_sb)
            pltpu.async_copy(table_hbm.at[idxt_v.at[j1]], rt_b, sem_tb)
            pltpu.make_async_copy(table_hbm.at[idxs_v.at[j0]],
                                  rs_a, sem_sa).wait()
            pltpu.sync_copy(rs_a, outs_hbm.at[pl.ds(row0, _CHUNK)])
            pltpu.make_async_copy(table_hbm.at[idxt_v.at[j0]],
                                  rt_a, sem_ta).wait()
            pltpu.sync_copy(rt_a, outt_hbm.at[pl.ds(row0, _CHUNK)])

            @pl.when(j0 + 2 < n_c)
            def _():
                pltpu.async_copy(table_hbm.at[idxs_v.at[j0 + 2]], rs_a, sem_sa)
                pltpu.async_copy(table_hbm.at[idxt_v.at[j0 + 2]], rt_a, sem_ta)

            pltpu.make_async_copy(table_hbm.at[idxs_v.at[j1]],
                                  rs_b, sem_sb).wait()
            pltpu.sync_copy(rs_b, outs_hbm.at[pl.ds(row1, _CHUNK)])
            pltpu.make_async_copy(table_hbm.at[idxt_v.at[j1]],
                                  rt_b, sem_tb).wait()
            pltpu.sync_copy(rt_b, outt_hbm.at[pl.ds(row1, _CHUNK)])
            return carry

        lax.fori_loop(0, n_c // 2, body, 0)

    return k(table, asrc4, atgt4)


def _tc_proj1(x, w1, dp0, dp1, n):
    def body(x_ref, w_ref, d0_ref, d1_ref, out_ref):
        deg = d0_ref[0:n, :] + d1_ref[0:n, :] + 1.0
        dinv = lax.rsqrt(deg)
        p = jnp.dot(x_ref[...], w_ref[...], preferred_element_type=_F32)
        out_ref[...] = p * dinv

    return pl.pallas_call(
        body,
        out_shape=jax.ShapeDtypeStruct((n, w1.shape[1]), _F32),
    )(x, w1, dp0, dp1)


def _tc_conv1_epilogue(a0, a1, hs1, dp0, dp1, w2, b1, n):
    def body(a0_ref, a1_ref, hs1_ref, d0_ref, d1_ref, w_ref, b_ref, out_ref):
        deg = d0_ref[0:n, :] + d1_ref[0:n, :] + 1.0
        dinv = lax.rsqrt(deg)
        agg = a0_ref[0:n, :] + a1_ref[0:n, :] + hs1_ref[...]
        h = jnp.maximum(agg * dinv + b_ref[...], 0.0)
        out_ref[...] = jnp.dot(h, w_ref[...], preferred_element_type=_F32) * dinv

    return pl.pallas_call(
        body,
        out_shape=jax.ShapeDtypeStruct((n, w2.shape[1]), _F32),
    )(a0, a1, hs1, dp0, dp1, w2, b1)


def _tc_heads_node(a0, a1, hs2, dp0, dp1, b2, p1, pb1, p2, pb2, armyf, n,
                   width):
    h = hs2.shape[1]
    pad = width - h - 1

    def body(a0_ref, a1_ref, hs2_ref, d0_ref, d1_ref, b2_ref,
             p1_ref, pb1_ref, p2_ref, pb2_ref, army_ref,
             plc_ref, table_ref):
        deg = d0_ref[0:n, :] + d1_ref[0:n, :] + 1.0
        dinv = lax.rsqrt(deg)
        agg = a0_ref[0:n, :] + a1_ref[0:n, :] + hs2_ref[...]
        emb = agg * dinv + b2_ref[...]
        hp = jnp.maximum(
            jnp.dot(emb, p1_ref[...], preferred_element_type=_F32)
            + pb1_ref[...], 0.0)
        plc_ref[...] = (jnp.dot(hp, p2_ref[...], preferred_element_type=_F32)
                        + pb2_ref[...])
        table_ref[...] = jnp.concatenate(
            [emb, army_ref[...], jnp.zeros((n, pad), _F32)], axis=1)

    return pl.pallas_call(
        body,
        out_shape=(jax.ShapeDtypeStruct((n, 1), _F32),
                   jax.ShapeDtypeStruct((n, width), _F32)),
    )(a0, a1, hs2, dp0, dp1, b2, p1, pb1, p2, pb2, armyf)


def _tc_edge_heads(s_tab, t_tab, asrc, atgt, ws, wt, bcat, es2, esb2,
                   as2, asb2, a_total, blk, h, max_army):
    grid = a_total // blk
    width = s_tab.shape[1]
    es_h = es2.shape[0]

    def body(s_ref, t_ref, asrc_ref, atgt_ref, ws_ref, wt_ref, bc_ref,
             es2_ref, esb2_ref, as2_ref, asb2_ref, el_ref, al_ref):
        s_emb = s_ref[:, 0:h]
        t_emb = t_ref[:, 0:h]
        sa = s_ref[:, h:h + 1]
        ta = t_ref[:, h:h + 1]
        hid = jnp.maximum(
            jnp.dot(s_emb, ws_ref[...], preferred_element_type=_F32)
            + jnp.dot(t_emb, wt_ref[...], preferred_element_type=_F32)
            + bc_ref[...], 0.0)
        el = (jnp.dot(hid[:, 0:es_h], es2_ref[...],
                      preferred_element_type=_F32) + esb2_ref[...])
        bad = jnp.logical_or(sa <= 2.0, ta >= 3.0 * sa).astype(_F32)
        selfe = (asrc_ref[...] == atgt_ref[...]).astype(_F32)
        el_ref[...] = el - bad - 100.0 * selfe
        al = (jnp.dot(hid[:, es_h:], as2_ref[...],
                      preferred_element_type=_F32) + asb2_ref[...])
        army_ix = lax.broadcasted_iota(jnp.int32, (blk, max_army), 1).astype(_F32)
        valid = army_ix <= sa - 1.0
        al_ref[...] = jnp.where(valid, al, _F32(-1e9))

    return pl.pallas_call(
        body,
        grid=(grid,),
        in_specs=[
            pl.BlockSpec((blk, width), lambda i: (i, 0)),
            pl.BlockSpec((blk, width), lambda i: (i, 0)),
            pl.BlockSpec((blk, 1), lambda i: (i, 0)),
            pl.BlockSpec((blk, 1), lambda i: (i, 0)),
            pl.BlockSpec(ws.shape, lambda i: (0, 0)),
            pl.BlockSpec(wt.shape, lambda i: (0, 0)),
            pl.BlockSpec(bcat.shape, lambda i: (0,)),
            pl.BlockSpec(es2.shape, lambda i: (0, 0)),
            pl.BlockSpec(esb2.shape, lambda i: (0,)),
            pl.BlockSpec(as2.shape, lambda i: (0, 0)),
            pl.BlockSpec(asb2.shape, lambda i: (0,)),
        ],
        out_specs=(
            pl.BlockSpec((blk, 1), lambda i: (i, 0)),
            pl.BlockSpec((blk, max_army), lambda i: (i, 0)),
        ),
        out_shape=(jax.ShapeDtypeStruct((a_total, 1), _F32),
                   jax.ShapeDtypeStruct((a_total, max_army), _F32)),
    )(s_tab, t_tab, asrc, atgt, ws, wt, bcat, es2, esb2, as2, asb2)


def kernel(x, edge_index, action_edges, army_counts, W1, b1, W2, b2,
           P1, pb1, P2, pb2, ES1, esb1, ES2, esb2, AS1, asb1, AS2, asb2):
    n, d = x.shape
    h = W1.shape[1]
    e = edge_index.shape[1]
    a = action_edges.shape[0]
    max_army = AS2.shape[1]
    width = 80

    n_acc = ((n + _NS * 8) + _NS * 8 - 1) // (_NS * 8) * (_NS * 8)
    ce = -(-e // (_NW * _CHUNK))
    ca = -(-a // (_NW * _CHUNK))
    ce0, ce1 = _split(2 * ce)
    ca0, ca1 = _split(2 * ca)
    a_pad = _NS * (ca0 + ca1) * _CHUNK

    src = edge_index[0].astype(jnp.int32)
    dst = edge_index[1].astype(jnp.int32)
    src3 = _pad_idx_split(src, 0, ce0, ce1)
    dst3 = _pad_idx_split(dst, n, ce0, ce1)
    asrc = action_edges[:, 0].astype(jnp.int32)
    atgt = action_edges[:, 1].astype(jnp.int32)
    asrc3 = _pad_idx_split(asrc, 0, ca0, ca1)
    atgt3 = _pad_idx_split(atgt, 0, ca0, ca1)

    ones_h = jnp.ones((_CHUNK,), _F32)
    zeros1 = jnp.zeros((n_acc,), _F32)
    zeros2 = jnp.zeros((n_acc, h), _F32)
    armyf = army_counts.astype(_F32).reshape(n, 1)

    degp = _sc_degree(dst3, ones_h, zeros1, n_acc, ce0, ce1)
    dp0 = degp[0].reshape(n_acc, 1)
    dp1 = degp[1].reshape(n_acc, 1)
    hs1 = _tc_proj1(x, W1, dp0, dp1, n)

    agg1 = _sc_aggregate(hs1, src3, dst3, zeros2, n_acc, ce0, ce1, h)
    hs2 = _tc_conv1_epilogue(agg1[0], agg1[1], hs1, dp0, dp1, W2, b1, n)

    agg2 = _sc_aggregate(hs2, src3, dst3, zeros2, n_acc, ce0, ce1, h)
    plc, table = _tc_heads_node(agg2[0], agg2[1], hs2, dp0, dp1, b2,
                                P1, pb1, P2, pb2, armyf, n, width)

    s_tab, t_tab = _sc_edge_gather(table, asrc3, atgt3, a_pad, ca0, ca1,
                                   width)
    ws = jnp.concatenate([ES1[:h], AS1[:h]], axis=1)
    wt = jnp.concatenate([ES1[h:], AS1[h:]], axis=1)
    bcat = jnp.concatenate([esb1, asb1])
    el, al = _tc_edge_heads(s_tab, t_tab, asrc.reshape(a, 1),
                            atgt.reshape(a, 1), ws, wt, bcat, ES2, esb2,
                            AS2, asb2, a, 2000, h, max_army)

    return plc.reshape(n), el.reshape(a), al


# speedup vs baseline: 1.0485x; 1.0108x over previous
import functools

import jax
import jax.numpy as jnp
from jax import lax
from jax.experimental import pallas as pl
from jax.experimental.pallas import tpu as pltpu
from jax.experimental.pallas import tpu_sc as plsc

_NC = 2
_NS = 16
_NW = _NC * _NS
_CHUNK = 128

_F32 = jnp.float32

_CORE0_SHARE = 0.75


def _split(total):
    n0 = int(round(total * _CORE0_SHARE / 2)) * 2
    n0 = min(max(n0, 2), total - 2)
    return n0, total - n0


def _pad_idx_split(idx, pad_value, n0, n1):
    idx = idx.astype(jnp.int32)
    total = _NS * (n0 + n1) * _CHUNK
    idx = jnp.concatenate(
        [idx, jnp.full((total - idx.shape[0],), pad_value, jnp.int32)])
    chunks = idx.reshape(_NS * (n0 + n1), _CHUNK)
    g0 = _NS * n0
    c0 = chunks[:g0].reshape(_NS, n0, _CHUNK)
    c1 = chunks[g0:].reshape(_NS, n1, _CHUNK)
    maxc = max(n0, n1)
    c0 = jnp.pad(c0, ((0, 0), (0, maxc - n0), (0, 0)))
    c1 = jnp.pad(c1, ((0, 0), (0, maxc - n1), (0, 0)))
    return jnp.stack([c0, c1])


def _sc_degree(dst4, ones_hbm, zeros1, n_acc, n0, n1):
    rows = n_acc // _NS
    maxc = max(n0, n1)
    mesh = plsc.VectorSubcoreMesh(core_axis_name="c", subcore_axis_name="s")

    @functools.partial(
        pl.kernel,
        out_type=jax.ShapeDtypeStruct((_NC, n_acc), _F32),
        mesh=mesh,
        scratch_types=[
            pltpu.VMEM((maxc, _CHUNK), jnp.int32),
            pltpu.VMEM((_CHUNK,), _F32),
            pltpu.VMEM_SHARED((n_acc,), _F32),
        ],
        compiler_params=pltpu.CompilerParams(use_tc_tiling_on_sc=False),
    )
    def k(dst_hbm, ones_h, zeros_h, out_hbm, idx_v, ones_v, acc):
        cid = lax.axis_index("c")
        sid = lax.axis_index("s")
        n_c = jnp.where(cid == 0, n0, n1)
        pltpu.sync_copy(zeros_h.at[pl.ds(sid * rows, rows)],
                        acc.at[pl.ds(sid * rows, rows)])
        pltpu.sync_copy(ones_h, ones_v)
        pltpu.sync_copy(dst_hbm.at[cid, sid], idx_v)
        plsc.subcore_barrier()

        def body(j, carry):
            pltpu.sync_copy(ones_v, acc.at[idx_v.at[j]], add=True)
            return carry

        lax.fori_loop(0, n_c, body, 0)
        plsc.subcore_barrier()
        pltpu.sync_copy(acc.at[pl.ds(sid * rows, rows)],
                        out_hbm.at[cid, pl.ds(sid * rows, rows)])

    return k(dst4, ones_hbm, zeros1)


def _sc_aggregate(hs, src4, dst4, zeros2, n_acc, n0, n1, h_dim):
    rows = n_acc // _NS
    maxc = max(n0, n1)
    mesh = plsc.VectorSubcoreMesh(core_axis_name="c", subcore_axis_name="s")

    @functools.partial(
        pl.kernel,
        out_type=jax.ShapeDtypeStruct((_NC, n_acc, h_dim), _F32),
        mesh=mesh,
        scratch_types=[
            pltpu.VMEM((maxc, _CHUNK), jnp.int32),
            pltpu.VMEM((maxc, _CHUNK), jnp.int32),
            pltpu.VMEM((_CHUNK, h_dim), _F32),
            pltpu.VMEM((_CHUNK, h_dim), _F32),
            pltpu.SemaphoreType.DMA,
            pltpu.SemaphoreType.DMA,
            pltpu.VMEM_SHARED((n_acc, h_dim), _F32),
        ],
        compiler_params=pltpu.CompilerParams(use_tc_tiling_on_sc=False),
    )
    def k(hs_hbm, src_hbm, dst_hbm, zeros_h, out_hbm,
          idxs_v, idxd_v, rows_a, rows_b, sem_a, sem_b, acc):
        cid = lax.axis_index("c")
        sid = lax.axis_index("s")
        n_c = jnp.where(cid == 0, n0, n1)
        pltpu.sync_copy(zeros_h.at[pl.ds(sid * rows, rows)],
                        acc.at[pl.ds(sid * rows, rows)])
        pltpu.sync_copy(src_hbm.at[cid, sid], idxs_v)
        pltpu.sync_copy(dst_hbm.at[cid, sid], idxd_v)
        plsc.subcore_barrier()

        pltpu.async_copy(hs_hbm.at[idxs_v.at[0]], rows_a, sem_a)

        def body(i, carry):
            j0 = 2 * i
            j1 = j0 + 1
            pltpu.async_copy(hs_hbm.at[idxs_v.at[j1]], rows_b, sem_b)
            pltpu.make_async_copy(hs_hbm.at[idxs_v.at[j0]],
                                  rows_a, sem_a).wait()
            pltpu.sync_copy(rows_a, acc.at[idxd_v.at[j0]], add=True)

            @pl.when(j0 + 2 < n_c)
            def _():
                pltpu.async_copy(hs_hbm.at[idxs_v.at[j0 + 2]], rows_a, sem_a)

            pltpu.make_async_copy(hs_hbm.at[idxs_v.at[j1]],
                                  rows_b, sem_b).wait()
            pltpu.sync_copy(rows_b, acc.at[idxd_v.at[j1]], add=True)
            return carry

        lax.fori_loop(0, n_c // 2, body, 0)
        plsc.subcore_barrier()
        pltpu.sync_copy(acc.at[pl.ds(sid * rows, rows)],
                        out_hbm.at[cid, pl.ds(sid * rows, rows)])

    return k(hs, src4, dst4, zeros2)


def _sc_edge_gather(table, asrc4, atgt4, a_pad, n0, n1, width):
    maxc = max(n0, n1)
    mesh = plsc.VectorSubcoreMesh(core_axis_name="c", subcore_axis_name="s")

    @functools.partial(
        pl.kernel,
        out_type=(jax.ShapeDtypeStruct((a_pad, width), _F32),
                  jax.ShapeDtypeStruct((a_pad, width), _F32)),
        mesh=mesh,
        scratch_types=[
            pltpu.VMEM((maxc, _CHUNK), jnp.int32),
            pltpu.VMEM((maxc, _CHUNK), jnp.int32),
            pltpu.VMEM((_CHUNK, width), _F32),
            pltpu.VMEM((_CHUNK, width), _F32),
            pltpu.VMEM((_CHUNK, width), _F32),
            pltpu.VMEM((_CHUNK, width), _F32),
            pltpu.SemaphoreType.DMA,
            pltpu.SemaphoreType.DMA,
            pltpu.SemaphoreType.DMA,
            pltpu.SemaphoreType.DMA,
        ],
        compiler_params=pltpu.CompilerParams(use_tc_tiling_on_sc=True),
    )
    def k(table_hbm, asrc_hbm, atgt_hbm, outs_hbm, outt_hbm,
          idxs_v, idxt_v, rs_a, rt_a, rs_b, rt_b,
          sem_sa, sem_ta, sem_sb, sem_tb):
        cid = lax.axis_index("c")
        sid = lax.axis_index("s")
        n_c = jnp.where(cid == 0, n0, n1)
        base = jnp.where(cid == 0, sid * n0, _NS * n0 + sid * n1) * _CHUNK
        pltpu.sync_copy(asrc_hbm.at[cid, sid], idxs_v)
        pltpu.sync_copy(atgt_hbm.at[cid, sid], idxt_v)

        pltpu.async_copy(table_hbm.at[idxs_v.at[0]], rs_a, sem_sa)
        pltpu.async_copy(table_hbm.at[idxt_v.at[0]], rt_a, sem_ta)

        def body(i, carry):
            j0 = 2 * i
            j1 = j0 + 1
            row0 = base + j0 * _CHUNK
            row1 = base + j1 * _CHUNK
            pltpu.async_copy(table_hbm.at[idxs_v.at[j1]], rs_b, sem_sb)
            pltpu.async_copy(table_hbm.at[idxt_v.at[j1]], rt_b, sem_tb)
            pltpu.make_async_copy(table_hbm.at[idxs_v.at[j0]],
                                  rs_a, sem_sa).wait()
            pltpu.sync_copy(rs_a, outs_hbm.at[pl.ds(row0, _CHUNK)])
            pltpu.make_async_copy(table_hbm.at[idxt_v.at[j0]],
                                  rt_a, sem_ta).wait()
            pltpu.sync_copy(rt_a, outt_hbm.at[pl.ds(row0, _CHUNK)])

            @pl.when(j0 + 2 < n_c)
            def _():
                pltpu.async_copy(table_hbm.at[idxs_v.at[j0 + 2]], rs_a, sem_sa)
                pltpu.async_copy(table_hbm.at[idxt_v.at[j0 + 2]], rt_a, sem_ta)

            pltpu.make_async_copy(table_hbm.at[idxs_v.at[j1]],
                                  rs_b, sem_sb).wait()
            pltpu.sync_copy(rs_b, outs_hbm.at[pl.ds(row1, _CHUNK)])
            pltpu.make_async_copy(table_hbm.at[idxt_v.at[j1]],
                                  rt_b, sem_tb).wait()
            pltpu.sync_copy(rt_b, outt_hbm.at[pl.ds(row1, _CHUNK)])
            return carry

        lax.fori_loop(0, n_c // 2, body, 0)

    return k(table, asrc4, atgt4)


def _tc_proj1(x, w1, dp0, dp1, n):
    def body(x_ref, w_ref, d0_ref, d1_ref, out_ref):
        deg = d0_ref[0:n, :] + d1_ref[0:n, :] + 1.0
        dinv = lax.rsqrt(deg)
        p = jnp.dot(x_ref[...], w_ref[...], preferred_element_type=_F32)
        out_ref[...] = p * dinv

    return pl.pallas_call(
        body,
        out_shape=jax.ShapeDtypeStruct((n, w1.shape[1]), _F32),
    )(x, w1, dp0, dp1)


def _tc_conv1_epilogue(a0, a1, hs1, dp0, dp1, w2, b1, n):
    def body(a0_ref, a1_ref, hs1_ref, d0_ref, d1_ref, w_ref, b_ref, out_ref):
        deg = d0_ref[0:n, :] + d1_ref[0:n, :] + 1.0
        dinv = lax.rsqrt(deg)
        agg = a0_ref[0:n, :] + a1_ref[0:n, :] + hs1_ref[...]
        h = jnp.maximum(agg * dinv + b_ref[...], 0.0)
        out_ref[...] = jnp.dot(h, w_ref[...], preferred_element_type=_F32) * dinv

    return pl.pallas_call(
        body,
        out_shape=jax.ShapeDtypeStruct((n, w2.shape[1]), _F32),
    )(a0, a1, hs1, dp0, dp1, w2, b1)


def _tc_heads_node(a0, a1, hs2, dp0, dp1, b2, p1, pb1, p2, pb2, armyf, n,
                   width):
    h = hs2.shape[1]
    pad = width - h - 1

    def body(a0_ref, a1_ref, hs2_ref, d0_ref, d1_ref, b2_ref,
             p1_ref, pb1_ref, p2_ref, pb2_ref, army_ref,
             plc_ref, table_ref):
        deg = d0_ref[0:n, :] + d1_ref[0:n, :] + 1.0
        dinv = lax.rsqrt(deg)
        agg = a0_ref[0:n, :] + a1_ref[0:n, :] + hs2_ref[...]
        emb = agg * dinv + b2_ref[...]
        hp = jnp.maximum(
            jnp.dot(emb, p1_ref[...], preferred_element_type=_F32)
            + pb1_ref[...], 0.0)
        plc_ref[...] = (jnp.dot(hp, p2_ref[...], preferred_element_type=_F32)
                        + pb2_ref[...])
        table_ref[...] = jnp.concatenate(
            [emb, army_ref[...], jnp.zeros((n, pad), _F32)], axis=1)

    return pl.pallas_call(
        body,
        out_shape=(jax.ShapeDtypeStruct((n, 1), _F32),
                   jax.ShapeDtypeStruct((n, width), _F32)),
    )(a0, a1, hs2, dp0, dp1, b2, p1, pb1, p2, pb2, armyf)


def _tc_edge_heads(s_tab, t_tab, asrc, atgt, ws, wt, bcat, es2, esb2,
                   as2, asb2, a_total, blk, h, max_army):
    grid = a_total // blk
    width = s_tab.shape[1]
    es_h = es2.shape[0]

    def body(s_ref, t_ref, asrc_ref, atgt_ref, ws_ref, wt_ref, bc_ref,
             es2_ref, esb2_ref, as2_ref, asb2_ref, el_ref, al_ref):
        s_emb = s_ref[:, 0:h]
        t_emb = t_ref[:, 0:h]
        sa = s_ref[:, h:h + 1]
        ta = t_ref[:, h:h + 1]
        hid = jnp.maximum(
            jnp.dot(s_emb, ws_ref[...], preferred_element_type=_F32)
            + jnp.dot(t_emb, wt_ref[...], preferred_element_type=_F32)
            + bc_ref[...], 0.0)
        el = (jnp.dot(hid[:, 0:es_h], es2_ref[...],
                      preferred_element_type=_F32) + esb2_ref[...])
        bad = jnp.logical_or(sa <= 2.0, ta >= 3.0 * sa).astype(_F32)
        selfe = (asrc_ref[...] == atgt_ref[...]).astype(_F32)
        el_ref[...] = el - bad - 100.0 * selfe
        al = (jnp.dot(hid[:, es_h:], as2_ref[...],
                      preferred_element_type=_F32) + asb2_ref[...])
        army_ix = lax.broadcasted_iota(jnp.int32, (blk, max_army), 1).astype(_F32)
        valid = army_ix <= sa - 1.0
        al_ref[...] = jnp.where(valid, al, _F32(-1e9))

    return pl.pallas_call(
        body,
        grid=(grid,),
        in_specs=[
            pl.BlockSpec((blk, width), lambda i: (i, 0)),
            pl.BlockSpec((blk, width), lambda i: (i, 0)),
            pl.BlockSpec((blk, 1), lambda i: (i, 0)),
            pl.BlockSpec((blk, 1), lambda i: (i, 0)),
            pl.BlockSpec(ws.shape, lambda i: (0, 0)),
            pl.BlockSpec(wt.shape, lambda i: (0, 0)),
            pl.BlockSpec(bcat.shape, lambda i: (0,)),
            pl.BlockSpec(es2.shape, lambda i: (0, 0)),
            pl.BlockSpec(esb2.shape, lambda i: (0,)),
            pl.BlockSpec(as2.shape, lambda i: (0, 0)),
            pl.BlockSpec(asb2.shape, lambda i: (0,)),
        ],
        out_specs=(
            pl.BlockSpec((blk, 1), lambda i: (i, 0)),
            pl.BlockSpec((blk, max_army), lambda i: (i, 0)),
        ),
        out_shape=(jax.ShapeDtypeStruct((a_total, 1), _F32),
                   jax.ShapeDtypeStruct((a_total, max_army), _F32)),
    )(s_tab, t_tab, asrc, atgt, ws, wt, bcat, es2, esb2, as2, asb2)


def kernel(x, edge_index, action_edges, army_counts, W1, b1, W2, b2,
           P1, pb1, P2, pb2, ES1, esb1, ES2, esb2, AS1, asb1, AS2, asb2):
    n, d = x.shape
    h = W1.shape[1]
    e = edge_index.shape[1]
    a = action_edges.shape[0]
    max_army = AS2.shape[1]
    width = 128

    n_acc = ((n + _NS * 8) + _NS * 8 - 1) // (_NS * 8) * (_NS * 8)
    ce = -(-e // (_NW * _CHUNK))
    ca = -(-a // (_NW * _CHUNK))
    ce0, ce1 = _split(2 * ce)
    ca0, ca1 = _split(2 * ca)
    a_pad = _NS * (ca0 + ca1) * _CHUNK

    src = edge_index[0].astype(jnp.int32)
    dst = edge_index[1].astype(jnp.int32)
    src3 = _pad_idx_split(src, 0, ce0, ce1)
    dst3 = _pad_idx_split(dst, n, ce0, ce1)
    asrc = action_edges[:, 0].astype(jnp.int32)
    atgt = action_edges[:, 1].astype(jnp.int32)
    asrc3 = _pad_idx_split(asrc, 0, ca0, ca1)
    atgt3 = _pad_idx_split(atgt, 0, ca0, ca1)

    ones_h = jnp.ones((_CHUNK,), _F32)
    zeros1 = jnp.zeros((n_acc,), _F32)
    zeros2 = jnp.zeros((n_acc, h), _F32)
    armyf = army_counts.astype(_F32).reshape(n, 1)

    degp = _sc_degree(dst3, ones_h, zeros1, n_acc, ce0, ce1)
    dp0 = degp[0].reshape(n_acc, 1)
    dp1 = degp[1].reshape(n_acc, 1)
    hs1 = _tc_proj1(x, W1, dp0, dp1, n)

    agg1 = _sc_aggregate(hs1, src3, dst3, zeros2, n_acc, ce0, ce1, h)
    hs2 = _tc_conv1_epilogue(agg1[0], agg1[1], hs1, dp0, dp1, W2, b1, n)

    agg2 = _sc_aggregate(hs2, src3, dst3, zeros2, n_acc, ce0, ce1, h)
    plc, table = _tc_heads_node(agg2[0], agg2[1], hs2, dp0, dp1, b2,
                                P1, pb1, P2, pb2, armyf, n, width)

    s_tab, t_tab = _sc_edge_gather(table, asrc3, atgt3, a_pad, ca0, ca1,
                                   width)
    ws = jnp.concatenate([ES1[:h], AS1[:h]], axis=1)
    wt = jnp.concatenate([ES1[h:], AS1[h:]], axis=1)
    bcat = jnp.concatenate([esb1, asb1])
    el, al = _tc_edge_heads(s_tab, t_tab, asrc.reshape(a, 1),
                            atgt.reshape(a, 1), ws, wt, bcat, ES2, esb2,
                            AS2, asb2, a, 2000, h, max_army)

    return plc.reshape(n), el.reshape(a), al


# speedup vs baseline: 1.1633x; 1.1095x over previous
import functools

import jax
import jax.numpy as jnp
from jax import lax
from jax.experimental import pallas as pl
from jax.experimental.pallas import tpu as pltpu
from jax.experimental.pallas import tpu_sc as plsc

_NC = 2
_NS = 16
_NW = _NC * _NS
_CHUNK = 128

_F32 = jnp.float32

_CORE0_SHARE = 0.75


def _split(total):
    n0 = int(round(total * _CORE0_SHARE / 2)) * 2
    n0 = min(max(n0, 2), total - 2)
    return n0, total - n0


def _pad_idx_split(idx, pad_value, n0, n1):
    idx = idx.astype(jnp.int32)
    total = _NS * (n0 + n1) * _CHUNK
    idx = jnp.concatenate(
        [idx, jnp.full((total - idx.shape[0],), pad_value, jnp.int32)])
    chunks = idx.reshape(_NS * (n0 + n1), _CHUNK)
    g0 = _NS * n0
    c0 = chunks[:g0].reshape(_NS, n0, _CHUNK)
    c1 = chunks[g0:].reshape(_NS, n1, _CHUNK)
    maxc = max(n0, n1)
    c0 = jnp.pad(c0, ((0, 0), (0, maxc - n0), (0, 0)))
    c1 = jnp.pad(c1, ((0, 0), (0, maxc - n1), (0, 0)))
    return jnp.stack([c0, c1])


def _sc_degree(dst4, ones_hbm, zeros1, n_acc, n0, n1):
    rows = n_acc // _NS
    maxc = max(n0, n1)
    mesh = plsc.VectorSubcoreMesh(core_axis_name="c", subcore_axis_name="s")

    @functools.partial(
        pl.kernel,
        out_type=jax.ShapeDtypeStruct((_NC, n_acc), _F32),
        mesh=mesh,
        scratch_types=[
            pltpu.VMEM((maxc, _CHUNK), jnp.int32),
            pltpu.VMEM((_CHUNK,), _F32),
            pltpu.VMEM_SHARED((n_acc,), _F32),
        ],
        compiler_params=pltpu.CompilerParams(use_tc_tiling_on_sc=False),
    )
    def k(dst_hbm, ones_h, zeros_h, out_hbm, idx_v, ones_v, acc):
        cid = lax.axis_index("c")
        sid = lax.axis_index("s")
        n_c = jnp.where(cid == 0, n0, n1)
        pltpu.sync_copy(zeros_h.at[pl.ds(sid * rows, rows)],
                        acc.at[pl.ds(sid * rows, rows)])
        pltpu.sync_copy(ones_h, ones_v)
        pltpu.sync_copy(dst_hbm.at[cid, sid], idx_v)
        plsc.subcore_barrier()

        def body(j, carry):
            pltpu.sync_copy(ones_v, acc.at[idx_v.at[j]], add=True)
            return carry

        lax.fori_loop(0, n_c, body, 0)
        plsc.subcore_barrier()
        pltpu.sync_copy(acc.at[pl.ds(sid * rows, rows)],
                        out_hbm.at[cid, pl.ds(sid * rows, rows)])

    return k(dst4, ones_hbm, zeros1)


def _sc_aggregate(hs, src4, dst4, zeros2, n_acc, n0, n1, h_dim):
    rows = n_acc // _NS
    maxc = max(n0, n1)
    mesh = plsc.VectorSubcoreMesh(core_axis_name="c", subcore_axis_name="s")

    @functools.partial(
        pl.kernel,
        out_type=jax.ShapeDtypeStruct((_NC, n_acc, h_dim), _F32),
        mesh=mesh,
        scratch_types=[
            pltpu.VMEM((maxc, _CHUNK), jnp.int32),
            pltpu.VMEM((maxc, _CHUNK), jnp.int32),
            pltpu.VMEM((_CHUNK, h_dim), _F32),
            pltpu.VMEM((_CHUNK, h_dim), _F32),
            pltpu.SemaphoreType.DMA,
            pltpu.SemaphoreType.DMA,
            pltpu.VMEM_SHARED((n_acc, h_dim), _F32),
        ],
        compiler_params=pltpu.CompilerParams(use_tc_tiling_on_sc=False),
    )
    def k(hs_hbm, src_hbm, dst_hbm, zeros_h, out_hbm,
          idxs_v, idxd_v, rows_a, rows_b, sem_a, sem_b, acc):
        cid = lax.axis_index("c")
        sid = lax.axis_index("s")
        n_c = jnp.where(cid == 0, n0, n1)
        pltpu.sync_copy(zeros_h.at[pl.ds(sid * rows, rows)],
                        acc.at[pl.ds(sid * rows, rows)])
        pltpu.sync_copy(src_hbm.at[cid, sid], idxs_v)
        pltpu.sync_copy(dst_hbm.at[cid, sid], idxd_v)
        plsc.subcore_barrier()

        pltpu.async_copy(hs_hbm.at[idxs_v.at[0]], rows_a, sem_a)

        def body(i, carry):
            j0 = 2 * i
            j1 = j0 + 1
            pltpu.async_copy(hs_hbm.at[idxs_v.at[j1]], rows_b, sem_b)
            pltpu.make_async_copy(hs_hbm.at[idxs_v.at[j0]],
                                  rows_a, sem_a).wait()
            pltpu.sync_copy(rows_a, acc.at[idxd_v.at[j0]], add=True)

            @pl.when(j0 + 2 < n_c)
            def _():
                pltpu.async_copy(hs_hbm.at[idxs_v.at[j0 + 2]], rows_a, sem_a)

            pltpu.make_async_copy(hs_hbm.at[idxs_v.at[j1]],
                                  rows_b, sem_b).wait()
            pltpu.sync_copy(rows_b, acc.at[idxd_v.at[j1]], add=True)
            return carry

        lax.fori_loop(0, n_c // 2, body, 0)
        plsc.subcore_barrier()
        pltpu.sync_copy(acc.at[pl.ds(sid * rows, rows)],
                        out_hbm.at[cid, pl.ds(sid * rows, rows)])

    return k(hs, src4, dst4, zeros2)


def _sc_edge_gather(table, asrc4, atgt4, a_pad, n0, n1, width):
    maxc = max(n0, n1)
    mesh = plsc.VectorSubcoreMesh(core_axis_name="c", subcore_axis_name="s")

    @functools.partial(
        pl.kernel,
        out_type=(jax.ShapeDtypeStruct((a_pad, width), _F32),
                  jax.ShapeDtypeStruct((a_pad, width), _F32)),
        mesh=mesh,
        scratch_types=[
            pltpu.VMEM((maxc, _CHUNK), jnp.int32),
            pltpu.VMEM((maxc, _CHUNK), jnp.int32),
            pltpu.VMEM((_CHUNK, width), _F32),
            pltpu.VMEM((_CHUNK, width), _F32),
            pltpu.VMEM((_CHUNK, width), _F32),
            pltpu.VMEM((_CHUNK, width), _F32),
            pltpu.SemaphoreType.DMA,
            pltpu.SemaphoreType.DMA,
            pltpu.SemaphoreType.DMA,
            pltpu.SemaphoreType.DMA,
        ],
        compiler_params=pltpu.CompilerParams(use_tc_tiling_on_sc=True),
    )
    def k(table_hbm, asrc_hbm, atgt_hbm, outs_hbm, outt_hbm,
          idxs_v, idxt_v, rs_a, rt_a, rs_b, rt_b,
          sem_sa, sem_ta, sem_sb, sem_tb):
        cid = lax.axis_index("c")
        sid = lax.axis_index("s")
        n_c = jnp.where(cid == 0, n0, n1)
        base = jnp.where(cid == 0, sid * n0, _NS * n0 + sid * n1) * _CHUNK
        pltpu.sync_copy(asrc_hbm.at[cid, sid], idxs_v)
        pltpu.sync_copy(atgt_hbm.at[cid, sid], idxt_v)

        pltpu.async_copy(table_hbm.at[idxs_v.at[0]], rs_a, sem_sa)
        pltpu.async_copy(table_hbm.at[idxt_v.at[0]], rt_a, sem_ta)

        def body(i, carry):
            j0 = 2 * i
            j1 = j0 + 1
            row0 = base + j0 * _CHUNK
            row1 = base + j1 * _CHUNK
            pltpu.async_copy(table_hbm.at[idxs_v.at[j1]], rs_b, sem_sb)
            pltpu.async_copy(table_hbm.at[idxt_v.at[j1]], rt_b, sem_tb)
            pltpu.make_async_copy(table_hbm.at[idxs_v.at[j0]],
                                  rs_a, sem_sa).wait()
            pltpu.sync_copy(rs_a, outs_hbm.at[pl.ds(row0, _CHUNK)])
            pltpu.make_async_copy(table_hbm.at[idxt_v.at[j0]],
                                  rt_a, sem_ta).wait()
            pltpu.sync_copy(rt_a, outt_hbm.at[pl.ds(row0, _CHUNK)])

            @pl.when(j0 + 2 < n_c)
            def _():
                pltpu.async_copy(table_hbm.at[idxs_v.at[j0 + 2]], rs_a, sem_sa)
                pltpu.async_copy(table_hbm.at[idxt_v.at[j0 + 2]], rt_a, sem_ta)

            pltpu.make_async_copy(table_hbm.at[idxs_v.at[j1]],
                                  rs_b, sem_sb).wait()
            pltpu.sync_copy(rs_b, outs_hbm.at[pl.ds(row1, _CHUNK)])
            pltpu.make_async_copy(table_hbm.at[idxt_v.at[j1]],
                                  rt_b, sem_tb).wait()
            pltpu.sync_copy(rt_b, outt_hbm.at[pl.ds(row1, _CHUNK)])
            return carry

        lax.fori_loop(0, n_c // 2, body, 0)

    return k(table, asrc4, atgt4)


def _tc_proj1(x, w1, dp0, dp1, n):
    def body(x_ref, w_ref, d0_ref, d1_ref, out_ref):
        deg = d0_ref[0:n, :] + d1_ref[0:n, :] + 1.0
        dinv = lax.rsqrt(deg)
        p = jnp.dot(x_ref[...], w_ref[...], preferred_element_type=_F32)
        out_ref[...] = p * dinv

    return pl.pallas_call(
        body,
        out_shape=jax.ShapeDtypeStruct((n, w1.shape[1]), _F32),
    )(x, w1, dp0, dp1)


def _tc_conv1_epilogue(a0, a1, hs1, dp0, dp1, w2, b1, n):
    def body(a0_ref, a1_ref, hs1_ref, d0_ref, d1_ref, w_ref, b_ref, out_ref):
        deg = d0_ref[0:n, :] + d1_ref[0:n, :] + 1.0
        dinv = lax.rsqrt(deg)
        agg = a0_ref[0:n, :] + a1_ref[0:n, :] + hs1_ref[...]
        h = jnp.maximum(agg * dinv + b_ref[...], 0.0)
        out_ref[...] = jnp.dot(h, w_ref[...], preferred_element_type=_F32) * dinv

    return pl.pallas_call(
        body,
        out_shape=jax.ShapeDtypeStruct((n, w2.shape[1]), _F32),
    )(a0, a1, hs1, dp0, dp1, w2, b1)


def _tc_heads_node(a0, a1, hs2, dp0, dp1, b2, p1, pb1, p2, pb2, armyf, n,
                   width):
    h = hs2.shape[1]
    pad = width - h - 2

    def body(a0_ref, a1_ref, hs2_ref, d0_ref, d1_ref, b2_ref,
             p1_ref, pb1_ref, p2_ref, pb2_ref, army_ref,
             plc_ref, table_ref):
        deg = d0_ref[0:n, :] + d1_ref[0:n, :] + 1.0
        dinv = lax.rsqrt(deg)
        agg = a0_ref[0:n, :] + a1_ref[0:n, :] + hs2_ref[...]
        emb = agg * dinv + b2_ref[...]
        hp = jnp.maximum(
            jnp.dot(emb, p1_ref[...], preferred_element_type=_F32)
            + pb1_ref[...], 0.0)
        plc_ref[...] = (jnp.dot(hp, p2_ref[...], preferred_element_type=_F32)
                        + pb2_ref[...])
        nid = lax.broadcasted_iota(jnp.int32, (n, 1), 0).astype(_F32)
        table_ref[...] = jnp.concatenate(
            [emb, army_ref[...], nid, jnp.zeros((n, pad), _F32)], axis=1)

    return pl.pallas_call(
        body,
        out_shape=(jax.ShapeDtypeStruct((n, 1), _F32),
                   jax.ShapeDtypeStruct((n, width), _F32)),
    )(a0, a1, hs2, dp0, dp1, b2, p1, pb1, p2, pb2, armyf)


def _tc_edge_heads(s_tab, t_tab, ws, wt, bcat, es2, esb2,
                   as2, asb2, a_total, blk, h, max_army):
    grid = a_total // blk
    width = s_tab.shape[1]
    es_h = es2.shape[0]

    def body(s_ref, t_ref, ws_ref, wt_ref, bc_ref,
             es2_ref, esb2_ref, as2_ref, asb2_ref, el_ref, al_ref):
        s_emb = s_ref[:, 0:h]
        t_emb = t_ref[:, 0:h]
        sa = s_ref[:, h:h + 1]
        ta = t_ref[:, h:h + 1]
        hid = jnp.maximum(
            jnp.dot(s_emb, ws_ref[...], preferred_element_type=_F32)
            + jnp.dot(t_emb, wt_ref[...], preferred_element_type=_F32)
            + bc_ref[...], 0.0)
        el = (jnp.dot(hid[:, 0:es_h], es2_ref[...],
                      preferred_element_type=_F32) + esb2_ref[...])
        bad = jnp.logical_or(sa <= 2.0, ta >= 3.0 * sa).astype(_F32)
        selfe = (s_ref[:, h + 1:h + 2] == t_ref[:, h + 1:h + 2]).astype(_F32)
        el_ref[...] = lax.transpose(el - bad - 100.0 * selfe, (1, 0))
        al = (jnp.dot(hid[:, es_h:], as2_ref[...],
                      preferred_element_type=_F32) + asb2_ref[...])
        army_ix = lax.broadcasted_iota(
            jnp.int32, (blk, max_army), 1).astype(_F32)
        valid = army_ix <= sa - 1.0
        al_ref[...] = lax.transpose(jnp.where(valid, al, _F32(-1e9)), (1, 0))

    return pl.pallas_call(
        body,
        grid=(grid,),
        in_specs=[
            pl.BlockSpec((blk, width), lambda i: (i, 0)),
            pl.BlockSpec((blk, width), lambda i: (i, 0)),
            pl.BlockSpec(ws.shape, lambda i: (0, 0)),
            pl.BlockSpec(wt.shape, lambda i: (0, 0)),
            pl.BlockSpec(bcat.shape, lambda i: (0,)),
            pl.BlockSpec(es2.shape, lambda i: (0, 0)),
            pl.BlockSpec(esb2.shape, lambda i: (0,)),
            pl.BlockSpec(as2.shape, lambda i: (0, 0)),
            pl.BlockSpec(asb2.shape, lambda i: (0,)),
        ],
        out_specs=(
            pl.BlockSpec((1, blk), lambda i: (0, i)),
            pl.BlockSpec((max_army, blk), lambda i: (0, i)),
        ),
        out_shape=(jax.ShapeDtypeStruct((1, a_total), _F32),
                   jax.ShapeDtypeStruct((max_army, a_total), _F32)),
    )(s_tab, t_tab, ws, wt, bcat, es2, esb2, as2, asb2)


def kernel(x, edge_index, action_edges, army_counts, W1, b1, W2, b2,
           P1, pb1, P2, pb2, ES1, esb1, ES2, esb2, AS1, asb1, AS2, asb2):
    n, d = x.shape
    h = W1.shape[1]
    e = edge_index.shape[1]
    a = action_edges.shape[0]
    max_army = AS2.shape[1]
    width = 128

    n_acc = ((n + _NS * 8) + _NS * 8 - 1) // (_NS * 8) * (_NS * 8)
    ce = -(-e // (_NW * _CHUNK))
    ca = -(-a // (_NW * _CHUNK))
    ce0, ce1 = _split(2 * ce)
    ca0, ca1 = _split(2 * ca)
    a_pad = _NS * (ca0 + ca1) * _CHUNK

    src = edge_index[0].astype(jnp.int32)
    dst = edge_index[1].astype(jnp.int32)
    src3 = _pad_idx_split(src, 0, ce0, ce1)
    dst3 = _pad_idx_split(dst, n, ce0, ce1)
    asrc = action_edges[:, 0].astype(jnp.int32)
    atgt = action_edges[:, 1].astype(jnp.int32)
    asrc3 = _pad_idx_split(asrc, 0, ca0, ca1)
    atgt3 = _pad_idx_split(atgt, 0, ca0, ca1)

    ones_h = jnp.ones((_CHUNK,), _F32)
    zeros1 = jnp.zeros((n_acc,), _F32)
    zeros2 = jnp.zeros((n_acc, h), _F32)
    armyf = army_counts.astype(_F32).reshape(n, 1)

    degp = _sc_degree(dst3, ones_h, zeros1, n_acc, ce0, ce1)
    dp0 = degp[0].reshape(n_acc, 1)
    dp1 = degp[1].reshape(n_acc, 1)
    hs1 = _tc_proj1(x, W1, dp0, dp1, n)

    agg1 = _sc_aggregate(hs1, src3, dst3, zeros2, n_acc, ce0, ce1, h)
    hs2 = _tc_conv1_epilogue(agg1[0], agg1[1], hs1, dp0, dp1, W2, b1, n)

    agg2 = _sc_aggregate(hs2, src3, dst3, zeros2, n_acc, ce0, ce1, h)
    plc, table = _tc_heads_node(agg2[0], agg2[1], hs2, dp0, dp1, b2,
                                P1, pb1, P2, pb2, armyf, n, width)

    s_tab, t_tab = _sc_edge_gather(table, asrc3, atgt3, a_pad, ca0, ca1,
                                   width)
    ws = jnp.concatenate([ES1[:h], AS1[:h]], axis=1)
    wt = jnp.concatenate([ES1[h:], AS1[h:]], axis=1)
    bcat = jnp.concatenate([esb1, asb1])
    el_t, al_t = _tc_edge_heads(s_tab, t_tab, ws, wt, bcat, ES2, esb2,
                                AS2, asb2, a, 3200, h, max_army)

    return plc.reshape(n), el_t.reshape(a), al_t.T
